# Initial kernel scaffold; baseline (speedup 1.0000x reference)
import jax
import jax.numpy as jnp
from jax.experimental import pallas as pl


def kernel(palettes, images):
    raise NotImplementedError("write your pallas kernel here")


# speedup vs baseline: 90.3257x; 90.3257x over previous
import functools

import jax
import jax.numpy as jnp
from jax import lax
from jax.experimental import pallas as pl
from jax.experimental.pallas import tpu as pltpu
from jax.experimental.pallas import tpu_sc as plsc

_B = 8
_K = 16
_C = 3
_H = 384
_W = 384
_P = _H * _W
_NW = 32
_WPB = _NW // _B
_QW = _P // _WPB
_CH = 6144
_NCH = _QW // _CH
_LANES = 16
_ALPHA = 0.001
_NPAIR = _K * (_K - 1) / 2.0


def _rsqrt(s):
    i = lax.bitcast_convert_type(s, jnp.int32)
    i = 0x5F3759DF - lax.shift_right_arithmetic(i, 1)
    y = lax.bitcast_convert_type(i, jnp.float32)
    for _ in range(3):
        y = y * (1.5 - 0.5 * s * y * y)
    return y


def _sc_body(img, palb, palv, out_px, out_pal, buf_r, buf_g, buf_b, palb_v,
             palv_v, stage_v, sem):
    cid = lax.axis_index("c")
    sid = lax.axis_index("s")
    wid = sid * 2 + cid
    b = wid // _WPB
    q = wid % _WPB

    pltpu.sync_copy(palb.at[b], palb_v)
    pr = [palb_v[pl.ds((0 * _K + k) * _LANES, _LANES)] for k in range(_K)]
    pg = [palb_v[pl.ds((1 * _K + k) * _LANES, _LANES)] for k in range(_K)]
    pb = [palb_v[pl.ds((2 * _K + k) * _LANES, _LANES)] for k in range(_K)]

    def chunk_compute(acc):
        def body(i, acc):
            base = i * _LANES
            r = buf_r[pl.ds(base, _LANES)]
            g = buf_g[pl.ds(base, _LANES)]
            bl = buf_b[pl.ds(base, _LANES)]
            m = None
            for k in range(_K):
                dr = r - pr[k]
                dg = g - pg[k]
                db = bl - pb[k]
                d = dr * dr + dg * dg + db * db
                m = d if m is None else jnp.minimum(m, d)
            return acc + m

        return lax.fori_loop(0, _CH // _LANES, body, acc)

    acc = jnp.zeros((_LANES,), jnp.float32)
    for ch in range(_NCH):
        start = (b * _C) * _P + q * _QW + ch * _CH
        pltpu.sync_copy(img.at[pl.ds(start, _CH)], buf_r)
        pltpu.sync_copy(img.at[pl.ds(start + _P, _CH)], buf_g)
        pltpu.sync_copy(img.at[pl.ds(start + 2 * _P, _CH)], buf_b)
        acc = chunk_compute(acc)

    stage_v[...] = acc
    pltpu.sync_copy(stage_v, out_px.at[wid])

    @pl.when(q == 0)
    def _():
        pltpu.sync_copy(palv.at[b], palv_v)
        lanes = lax.iota(jnp.int32, _LANES)
        pv0 = palv_v[pl.ds(0 * _LANES, _LANES)]
        pv1 = palv_v[pl.ds(1 * _LANES, _LANES)]
        pv2 = palv_v[pl.ds(2 * _LANES, _LANES)]
        pal_acc = jnp.zeros((_LANES,), jnp.float32)
        for j in range(_K):
            dr = pv0 - pr[j]
            dg = pv1 - pg[j]
            db = pv2 - pb[j]
            d2 = dr * dr + dg * dg + db * db
            dist = d2 * _rsqrt(d2)
            mask = jnp.where(lanes > j, 1.0, 0.0).astype(jnp.float32)
            pal_acc = pal_acc + dist * mask
        stage_v[...] = pal_acc
        pltpu.sync_copy(stage_v, out_pal.at[b])


_sc_kernel = functools.partial(
    pl.kernel,
    out_type=[
        jax.ShapeDtypeStruct((_NW, _LANES), jnp.float32),
        jax.ShapeDtypeStruct((_B, _LANES), jnp.float32),
    ],
    mesh=plsc.VectorSubcoreMesh(core_axis_name="c", subcore_axis_name="s"),
    scratch_types=[
        pltpu.VMEM((_CH,), jnp.float32),
        pltpu.VMEM((_CH,), jnp.float32),
        pltpu.VMEM((_CH,), jnp.float32),
        pltpu.VMEM((_C * _K * _LANES,), jnp.float32),
        pltpu.VMEM((_C * _LANES,), jnp.float32),
        pltpu.VMEM((_LANES,), jnp.float32),
        pltpu.SemaphoreType.DMA,
    ],
)(_sc_body)


@jax.jit
def kernel(palettes, images):
    palv = jnp.transpose(palettes, (0, 2, 1))
    palb = jnp.broadcast_to(palv[..., None], (_B, _C, _K, _LANES))
    palb = palb.reshape(_B, _C * _K * _LANES)
    img = images.reshape(_B * _C * _P)
    out_px, out_pal = _sc_kernel(img, palb, palv.reshape(_B, _C * _K))
    mse = jnp.sum(out_px) / (_B * _C * _P)
    pal = jnp.sum(out_pal) / (_NPAIR * _B)
    return mse - _ALPHA * pal
